# baseline (speedup 1.0000x reference)
import functools

import jax
import jax.numpy as jnp
from jax import lax
from jax.experimental import pallas as pl
from jax.experimental.pallas import tpu as pltpu
from jax.experimental.pallas import tpu_sc as plsc

N = 100000
E = 3200000
NUM_GRAPHS = 512
HID = 20
DR = 10
DH = 16

NC, NS = 2, 16
NW = NC * NS
CH = 512
EPAD = 3211264
EPT1 = EPAD // NW
EPT2 = EPAD // NS
NPAD = N + 96
RINIT = NPAD // NS

PPAD = 131072
GPAD = 640
GINIT = GPAD // NS


def _make_agg(split_edges, ept, acc_rows, rinit):
  nch = ept // CH
  nt = nch // 2

  @functools.partial(
      pl.kernel,
      out_type=(
          jax.ShapeDtypeStruct((acc_rows, DH), jnp.float32),
          jax.ShapeDtypeStruct((acc_rows, DH), jnp.float32),
      ),
      mesh=plsc.VectorSubcoreMesh(core_axis_name="c", subcore_axis_name="s",
                                  num_cores=NC, num_subcores=NS),
      compiler_params=pltpu.CompilerParams(use_tc_tiling_on_sc=False),
      scratch_types=[
          pltpu.VMEM_SHARED((acc_rows, DH), jnp.float32),
          pltpu.VMEM((CH,), jnp.int32),
          pltpu.VMEM((CH,), jnp.int32),
          pltpu.VMEM((CH,), jnp.int32),
          pltpu.VMEM((CH,), jnp.int32),
          pltpu.VMEM((CH, DH), jnp.float32),
          pltpu.VMEM((CH, DH), jnp.float32),
          pltpu.SemaphoreType.DMA,
          pltpu.SemaphoreType.DMA,
          pltpu.SemaphoreType.DMA,
          pltpu.SemaphoreType.DMA,
      ],
  )
  def agg(ta, tb, ia, ib, srcf, dstf, oa, ob, acc, sv0, dv0, sv1, dv1,
          r0, r1, g0, g1, s0, s1):
    cid = lax.axis_index("c")
    sid = lax.axis_index("s")
    ebase = ((sid * NC + cid) if split_edges else sid) * ept

    def drain(sem, rbuf, table):
      pltpu.make_async_copy(table.at[pl.ds(0, CH)], rbuf, sem).wait()

    def run(table, initr, out):
      pltpu.sync_copy(initr.at[pl.ds(sid * rinit, rinit)],
                      acc.at[pl.ds(sid * rinit, rinit)])
      plsc.subcore_barrier()

      pltpu.sync_copy(srcf.at[pl.ds(ebase, CH)], sv0)
      pltpu.sync_copy(dstf.at[pl.ds(ebase, CH)], dv0)
      pltpu.async_copy(table.at[sv0], r0, g0)

      def pair(t, carry):
        off = ebase + 2 * t * CH

        @pl.when(t > 0)
        def _():
          drain(s1, r1, table)
        pltpu.sync_copy(srcf.at[pl.ds(off + CH, CH)], sv1)
        pltpu.sync_copy(dstf.at[pl.ds(off + CH, CH)], dv1)
        pltpu.async_copy(table.at[sv1], r1, g1)

        drain(g0, r0, table)
        pltpu.async_copy(r0, acc.at[dv0], s0, add=True)

        @pl.when(t + 1 < nt)
        def _():
          drain(s0, r0, table)
          pltpu.sync_copy(srcf.at[pl.ds(off + 2 * CH, CH)], sv0)
          pltpu.sync_copy(dstf.at[pl.ds(off + 2 * CH, CH)], dv0)
          pltpu.async_copy(table.at[sv0], r0, g0)

        drain(g1, r1, table)
        pltpu.async_copy(r1, acc.at[dv1], s1, add=True)
        return carry

      lax.fori_loop(0, nt, pair, 0)
      drain(s0, r0, table)
      drain(s1, r1, table)
      plsc.subcore_barrier()
      pltpu.sync_copy(acc.at[pl.ds(sid * rinit, rinit)],
                      out.at[pl.ds(sid * rinit, rinit)])

    @pl.when(cid == 0)
    def _():
      run(ta, ia, oa)

    @pl.when(cid == 1)
    def _():
      run(tb, ib, ob)

  return agg


RB = 5000
NBLK = N // RB


def _mlpbn(groups, w1, b1, w2, b2, g, bb, z2buf):
  sizes = [len(gr) for gr in groups]
  ng = len(groups)
  nin = sum(sizes)

  def body(*refs):
    in_refs = refs[:nin]
    (w1_ref, b1_ref, w2_ref, b2_ref, g_ref, bb_ref, z2in_ref,
     z2o_ref, ha_ref, hb_ref, st) = refs[nin:]
    i = pl.program_id(0)

    @pl.when(i == 0)
    def _():
      st[...] = jnp.zeros_like(st)

    @pl.when(i < NBLK)
    def _():
      parts = []
      k = 0
      for sz in sizes:
        acc = in_refs[k][...]
        for j in range(1, sz):
          acc = acc + in_refs[k + j][...]
        parts.append(acc)
        k += sz
      z = parts[0] if ng == 1 else jnp.concatenate(parts, axis=1)
      a = jnp.maximum(
          jnp.dot(z, w1_ref[...], preferred_element_type=jnp.float32)
          + b1_ref[...], 0.0)
      z2 = jnp.dot(a, w2_ref[...], preferred_element_type=jnp.float32) \
          + b2_ref[...]
      z2o_ref[...] = z2
      s1 = jnp.sum(z2, axis=0, keepdims=True)
      s2 = jnp.sum(z2 * z2, axis=0, keepdims=True)
      st[...] += jnp.concatenate([s1, s2], axis=0)

    @pl.when(i >= NBLK)
    def _():
      stv = st[...]
      m = stv[0:1, :] * (1.0 / N)
      v = stv[1:2, :] * (1.0 / N) - m * m
      scale = g_ref[...] * lax.rsqrt(v + 1e-5)
      shift = bb_ref[...] - m * scale
      res = jnp.maximum(z2in_ref[...] * scale + shift, 0.0)
      zpad = jnp.zeros((res.shape[0], DH - DR), jnp.float32)
      ha_ref[...] = jnp.concatenate([res[:, 0:DR], zpad], axis=1)
      hb_ref[...] = jnp.concatenate([res[:, DR:HID], zpad], axis=1)

  din = ng * DH
  flat = [a for gr in groups for a in gr]
  lo = lambda i: (jnp.minimum(i, NBLK - 1), 0)
  hi = lambda i: (jnp.maximum(i - NBLK, 0), 0)
  cst = lambda i: (0, 0)
  _, ha, hb = pl.pallas_call(
      body,
      grid=(2 * NBLK,),
      in_specs=[pl.BlockSpec((RB, DH), lo)] * nin + [
          pl.BlockSpec((din, HID), cst),
          pl.BlockSpec((1, HID), cst),
          pl.BlockSpec((HID, HID), cst),
          pl.BlockSpec((1, HID), cst),
          pl.BlockSpec((1, HID), cst),
          pl.BlockSpec((1, HID), cst),
          pl.BlockSpec((RB, HID), hi),
      ],
      out_specs=[
          pl.BlockSpec((RB, HID), lo),
          pl.BlockSpec((RB, DH), hi),
          pl.BlockSpec((RB, DH), hi),
      ],
      out_shape=[
          jax.ShapeDtypeStruct((N, HID), jnp.float32),
          jax.ShapeDtypeStruct((NPAD, DH), jnp.float32),
          jax.ShapeDtypeStruct((NPAD, DH), jnp.float32),
      ],
      scratch_shapes=[pltpu.VMEM((2, HID), jnp.float32)],
      input_output_aliases={nin + 6: 0},
  )(*flat, w1, b1, w2, b2, g, bb, z2buf)
  return ha, hb


def _fc(qa, qb, w, b):
  def body(qa_ref, qb_ref, w_ref, b_ref, o_ref):
    q = jnp.concatenate([qa_ref[...], qb_ref[...]], axis=1)
    o_ref[...] = jnp.dot(q, w_ref[...],
                         preferred_element_type=jnp.float32) + b_ref[...]

  return pl.pallas_call(
      body,
      grid=(1,),
      in_specs=[pl.BlockSpec((NUM_GRAPHS, DH), lambda i: (0, 0))] * 2 + [
          pl.BlockSpec((2 * DH, 2), lambda i: (0, 0)),
          pl.BlockSpec((1, 2), lambda i: (0, 0)),
      ],
      out_specs=pl.BlockSpec((NUM_GRAPHS, 2), lambda i: (0, 0)),
      out_shape=jax.ShapeDtypeStruct((NUM_GRAPHS, 2), jnp.float32),
  )(qa, qb, w, b)


def kernel(x, edge_index, batch,
           c0_W1, c0_b1, c0_W2, c0_b2,
           c1_W1, c1_b1, c1_W2, c1_b2,
           c2_W1, c2_b1, c2_W2, c2_b2,
           bn0_g, bn0_b, bn1_g, bn1_b, bn2_g, bn2_b,
           fc_W, fc_b):
  src = edge_index[0]
  dst = edge_index[1]
  epad = EPAD - E
  srcf = jnp.concatenate([src, jnp.zeros((epad,), jnp.int32)])
  dpad = N + jnp.arange(epad, dtype=jnp.int32) % (NPAD - N)
  dstf = jnp.concatenate([dst, dpad])
  zn = jnp.zeros((NPAD, DH), jnp.float32)

  zw = jnp.zeros((DH - DR, HID), jnp.float32)
  c0_W1p = jnp.concatenate([c0_W1, zw], axis=0)
  c1_W1p = jnp.concatenate([c1_W1[:DR], zw, c1_W1[DR:], zw], axis=0)
  c2_W1p = jnp.concatenate([c2_W1[:DR], zw, c2_W1[DR:], zw], axis=0)
  zf = jnp.zeros((DH - DR, 2), jnp.float32)
  fc_Wp = jnp.concatenate([fc_W[:DR], zf, fc_W[DR:], zf], axis=0)

  agg1 = _make_agg(True, EPT1, NPAD, RINIT)
  agg2 = _make_agg(False, EPT2, NPAD, RINIT)

  xp = jnp.pad(x, ((0, NPAD - N), (0, DH - DR)))
  z2buf = jnp.zeros((N, HID), jnp.float32)
  p0, p1 = agg1(xp, xp, xp, zn, srcf, dstf)
  ha, hb = _mlpbn([[p0, p1]], c0_W1p, c0_b1.reshape(1, HID),
                  c0_W2, c0_b2.reshape(1, HID),
                  bn0_g.reshape(1, HID), bn0_b.reshape(1, HID), z2buf)

  for (w1, b1, w2, b2, g, bb) in (
      (c1_W1p, c1_b1, c1_W2, c1_b2, bn1_g, bn1_b),
      (c2_W1p, c2_b1, c2_W2, c2_b2, bn2_g, bn2_b),
  ):
    pa, pb = agg2(ha, hb, ha, hb, srcf, dstf)
    ha, hb = _mlpbn([[pa], [pb]], w1, b1.reshape(1, HID),
                    w2, b2.reshape(1, HID),
                    g.reshape(1, HID), bb.reshape(1, HID), z2buf)

  ppad = PPAD - N
  ridx = jnp.concatenate([jnp.arange(N, dtype=jnp.int32),
                          jnp.zeros((ppad,), jnp.int32)])
  bpad = NUM_GRAPHS + jnp.arange(ppad, dtype=jnp.int32) % (GPAD - NUM_GRAPHS)
  bidx = jnp.concatenate([batch, bpad])
  zg = jnp.zeros((GPAD, DH), jnp.float32)
  poolk = _make_agg(False, PPAD // NS, GPAD, GINIT)
  qa, qb = poolk(ha, hb, zg, zg, ridx, bidx)
  return _fc(qa[:NUM_GRAPHS], qb[:NUM_GRAPHS], fc_Wp, fc_b.reshape(1, 2))


# speedup vs baseline: 1.0325x; 1.0325x over previous
import functools

import jax
import jax.numpy as jnp
from jax import lax
from jax.experimental import pallas as pl
from jax.experimental.pallas import tpu as pltpu
from jax.experimental.pallas import tpu_sc as plsc

N = 100000
E = 3200000
NUM_GRAPHS = 512
HID = 20
DR = 10
DH = 16

NC, NS = 2, 16
NW = NC * NS
CH = 512
EPAD = 3211264
EPT1 = EPAD // NW
EPT2 = EPAD // NS
NPAD = N + 96
RINIT = NPAD // NS

PPAD = 131072
GPAD = 640
GINIT = GPAD // NS


def _make_agg(split_edges, ept, acc_rows, rinit):
  nch = ept // CH
  nt = nch // 2

  @functools.partial(
      pl.kernel,
      out_type=(
          jax.ShapeDtypeStruct((acc_rows, DH), jnp.float32),
          jax.ShapeDtypeStruct((acc_rows, DH), jnp.float32),
      ),
      mesh=plsc.VectorSubcoreMesh(core_axis_name="c", subcore_axis_name="s",
                                  num_cores=NC, num_subcores=NS),
      compiler_params=pltpu.CompilerParams(use_tc_tiling_on_sc=False),
      scratch_types=[
          pltpu.VMEM_SHARED((acc_rows, DH), jnp.float32),
          pltpu.VMEM((CH,), jnp.int32),
          pltpu.VMEM((CH,), jnp.int32),
          pltpu.VMEM((CH,), jnp.int32),
          pltpu.VMEM((CH,), jnp.int32),
          pltpu.VMEM((CH, DH), jnp.float32),
          pltpu.VMEM((CH, DH), jnp.float32),
          pltpu.SemaphoreType.DMA,
          pltpu.SemaphoreType.DMA,
          pltpu.SemaphoreType.DMA,
          pltpu.SemaphoreType.DMA,
      ],
  )
  def agg(ta, tb, ia, ib, srcf, dstf, oa, ob, acc, sv0, dv0, sv1, dv1,
          r0, r1, g0, g1, s0, s1):
    cid = lax.axis_index("c")
    sid = lax.axis_index("s")
    ebase = ((sid * NC + cid) if split_edges else sid) * ept

    def drain(sem, rbuf, table):
      pltpu.make_async_copy(table.at[pl.ds(0, CH)], rbuf, sem).wait()

    def run(table, initr, out):
      pltpu.sync_copy(initr.at[pl.ds(sid * rinit, rinit)],
                      acc.at[pl.ds(sid * rinit, rinit)])
      plsc.subcore_barrier()

      pltpu.sync_copy(srcf.at[pl.ds(ebase, CH)], sv0)
      pltpu.sync_copy(dstf.at[pl.ds(ebase, CH)], dv0)
      pltpu.async_copy(table.at[sv0], r0, g0)

      def pair(t, carry):
        off = ebase + 2 * t * CH

        @pl.when(t > 0)
        def _():
          drain(s1, r1, table)
        pltpu.sync_copy(srcf.at[pl.ds(off + CH, CH)], sv1)
        pltpu.sync_copy(dstf.at[pl.ds(off + CH, CH)], dv1)
        pltpu.async_copy(table.at[sv1], r1, g1)

        drain(g0, r0, table)
        pltpu.async_copy(r0, acc.at[dv0], s0, add=True)

        @pl.when(t + 1 < nt)
        def _():
          drain(s0, r0, table)
          pltpu.sync_copy(srcf.at[pl.ds(off + 2 * CH, CH)], sv0)
          pltpu.sync_copy(dstf.at[pl.ds(off + 2 * CH, CH)], dv0)
          pltpu.async_copy(table.at[sv0], r0, g0)

        drain(g1, r1, table)
        pltpu.async_copy(r1, acc.at[dv1], s1, add=True)
        return carry

      lax.fori_loop(0, nt, pair, 0)
      drain(s0, r0, table)
      drain(s1, r1, table)
      plsc.subcore_barrier()
      pltpu.sync_copy(acc.at[pl.ds(sid * rinit, rinit)],
                      out.at[pl.ds(sid * rinit, rinit)])

    @pl.when(cid == 0)
    def _():
      run(ta, ia, oa)

    @pl.when(cid == 1)
    def _():
      run(tb, ib, ob)

  return agg


RB = 10000
NBLK = N // RB


def _mlp(groups, w1, b1, w2, b2):
  sizes = [len(g) for g in groups]
  ng = len(groups)
  nin = sum(sizes)

  def body(*refs):
    in_refs = refs[:nin]
    w1_ref, b1_ref, w2_ref, b2_ref, z2_ref, st_ref = refs[nin:]
    parts = []
    k = 0
    for sz in sizes:
      acc = in_refs[k][...]
      for j in range(1, sz):
        acc = acc + in_refs[k + j][...]
      parts.append(acc)
      k += sz
    z = parts[0] if ng == 1 else jnp.concatenate(parts, axis=1)
    a = jnp.maximum(
        jnp.dot(z, w1_ref[...], preferred_element_type=jnp.float32)
        + b1_ref[...], 0.0)
    z2 = jnp.dot(a, w2_ref[...], preferred_element_type=jnp.float32) \
        + b2_ref[...]
    z2_ref[...] = z2

    @pl.when(pl.program_id(0) == 0)
    def _():
      st_ref[...] = jnp.zeros_like(st_ref)

    s1 = jnp.sum(z2, axis=0, keepdims=True)
    s2 = jnp.sum(z2 * z2, axis=0, keepdims=True)
    st_ref[...] += jnp.concatenate([s1, s2], axis=0)

  din = ng * DH
  flat = [a for g in groups for a in g]
  return pl.pallas_call(
      body,
      grid=(NBLK,),
      in_specs=[pl.BlockSpec((RB, DH), lambda i: (i, 0))] * nin + [
          pl.BlockSpec((din, HID), lambda i: (0, 0)),
          pl.BlockSpec((1, HID), lambda i: (0, 0)),
          pl.BlockSpec((HID, HID), lambda i: (0, 0)),
          pl.BlockSpec((1, HID), lambda i: (0, 0)),
      ],
      out_specs=[
          pl.BlockSpec((RB, HID), lambda i: (i, 0)),
          pl.BlockSpec((2, HID), lambda i: (0, 0)),
      ],
      out_shape=[
          jax.ShapeDtypeStruct((N, HID), jnp.float32),
          jax.ShapeDtypeStruct((2, HID), jnp.float32),
      ],
  )(*flat, w1, b1, w2, b2)


def _bnrelu(z2, st, g, b):
  def body(z2_ref, st_ref, g_ref, b_ref, oa_ref, ob_ref):
    st = st_ref[...]
    m = st[0:1, :] * (1.0 / N)
    v = st[1:2, :] * (1.0 / N) - m * m
    scale = g_ref[...] * lax.rsqrt(v + 1e-5)
    shift = b_ref[...] - m * scale
    res = jnp.maximum(z2_ref[...] * scale + shift, 0.0)
    zpad = jnp.zeros((res.shape[0], DH - DR), jnp.float32)
    oa_ref[...] = jnp.concatenate([res[:, 0:DR], zpad], axis=1)
    ob_ref[...] = jnp.concatenate([res[:, DR:HID], zpad], axis=1)

  return pl.pallas_call(
      body,
      grid=(NBLK,),
      in_specs=[
          pl.BlockSpec((RB, HID), lambda i: (i, 0)),
          pl.BlockSpec((2, HID), lambda i: (0, 0)),
          pl.BlockSpec((1, HID), lambda i: (0, 0)),
          pl.BlockSpec((1, HID), lambda i: (0, 0)),
      ],
      out_specs=[
          pl.BlockSpec((RB, DH), lambda i: (i, 0)),
          pl.BlockSpec((RB, DH), lambda i: (i, 0)),
      ],
      out_shape=[
          jax.ShapeDtypeStruct((NPAD, DH), jnp.float32),
          jax.ShapeDtypeStruct((NPAD, DH), jnp.float32),
      ],
  )(z2, st, g, b)


def _fc(qa, qb, w, b):
  def body(qa_ref, qb_ref, w_ref, b_ref, o_ref):
    q = jnp.concatenate([qa_ref[...], qb_ref[...]], axis=1)
    o_ref[...] = jnp.dot(q, w_ref[...],
                         preferred_element_type=jnp.float32) + b_ref[...]

  return pl.pallas_call(
      body,
      grid=(1,),
      in_specs=[pl.BlockSpec((NUM_GRAPHS, DH), lambda i: (0, 0))] * 2 + [
          pl.BlockSpec((2 * DH, 2), lambda i: (0, 0)),
          pl.BlockSpec((1, 2), lambda i: (0, 0)),
      ],
      out_specs=pl.BlockSpec((NUM_GRAPHS, 2), lambda i: (0, 0)),
      out_shape=jax.ShapeDtypeStruct((NUM_GRAPHS, 2), jnp.float32),
  )(qa, qb, w, b)


def kernel(x, edge_index, batch,
           c0_W1, c0_b1, c0_W2, c0_b2,
           c1_W1, c1_b1, c1_W2, c1_b2,
           c2_W1, c2_b1, c2_W2, c2_b2,
           bn0_g, bn0_b, bn1_g, bn1_b, bn2_g, bn2_b,
           fc_W, fc_b):
  src = edge_index[0]
  dst = edge_index[1]
  epad = EPAD - E
  srcf = jnp.concatenate([src, jnp.zeros((epad,), jnp.int32)])
  dpad = N + jnp.arange(epad, dtype=jnp.int32) % (NPAD - N)
  dstf = jnp.concatenate([dst, dpad])
  zn = jnp.zeros((NPAD, DH), jnp.float32)

  zw = jnp.zeros((DH - DR, HID), jnp.float32)
  c0_W1p = jnp.concatenate([c0_W1, zw], axis=0)
  c1_W1p = jnp.concatenate([c1_W1[:DR], zw, c1_W1[DR:], zw], axis=0)
  c2_W1p = jnp.concatenate([c2_W1[:DR], zw, c2_W1[DR:], zw], axis=0)
  zf = jnp.zeros((DH - DR, 2), jnp.float32)
  fc_Wp = jnp.concatenate([fc_W[:DR], zf, fc_W[DR:], zf], axis=0)

  agg1 = _make_agg(True, EPT1, NPAD, RINIT)
  agg2 = _make_agg(False, EPT2, NPAD, RINIT)

  xp = jnp.pad(x, ((0, NPAD - N), (0, DH - DR)))
  p0, p1 = agg1(xp, xp, xp, zn, srcf, dstf)
  z2, st = _mlp([[p0, p1]], c0_W1p, c0_b1.reshape(1, HID),
                c0_W2, c0_b2.reshape(1, HID))
  ha, hb = _bnrelu(z2, st, bn0_g.reshape(1, HID), bn0_b.reshape(1, HID))

  for (w1, b1, w2, b2, g, bb) in (
      (c1_W1p, c1_b1, c1_W2, c1_b2, bn1_g, bn1_b),
      (c2_W1p, c2_b1, c2_W2, c2_b2, bn2_g, bn2_b),
  ):
    pa, pb = agg2(ha, hb, ha, hb, srcf, dstf)
    z2, st = _mlp([[pa], [pb]], w1, b1.reshape(1, HID),
                  w2, b2.reshape(1, HID))
    ha, hb = _bnrelu(z2, st, g.reshape(1, HID), bb.reshape(1, HID))

  ppad = PPAD - N
  ridx = jnp.concatenate([jnp.arange(N, dtype=jnp.int32),
                          jnp.zeros((ppad,), jnp.int32)])
  bpad = NUM_GRAPHS + jnp.arange(ppad, dtype=jnp.int32) % (GPAD - NUM_GRAPHS)
  bidx = jnp.concatenate([batch, bpad])
  zg = jnp.zeros((GPAD, DH), jnp.float32)
  poolk = _make_agg(False, PPAD // NS, GPAD, GINIT)
  qa, qb = poolk(ha, hb, zg, zg, ridx, bidx)
  return _fc(qa[:NUM_GRAPHS], qb[:NUM_GRAPHS], fc_Wp, fc_b.reshape(1, 2))
